# baseline (speedup 1.0000x reference)
import jax
import jax.numpy as jnp
from jax import lax
from jax.experimental import pallas as pl
from jax.experimental.pallas import tpu as pltpu
from jax.experimental.pallas import tpu_sc as plsc

_NC = 2
_NS = 16
_NW = _NC * _NS
_CHUNK = 128


def kernel(idx, W):
    B = idx.shape[0]
    NZ = W.shape[1]
    b_per_w = B // _NW
    n_chunks = b_per_w // _CHUNK

    mesh = plsc.VectorSubcoreMesh(core_axis_name="c", subcore_axis_name="s")

    @pl.kernel(
        mesh=mesh,
        out_type=jax.ShapeDtypeStruct((B, NZ), W.dtype),
        compiler_params=pltpu.CompilerParams(
            needs_layout_passes=False, use_tc_tiling_on_sc=False),
        scratch_types=[
            pltpu.VMEM((b_per_w,), jnp.int32),
            pltpu.VMEM((b_per_w, NZ), jnp.float32),
            pltpu.SemaphoreType.DMA,
        ],
    )
    def k(idx_hbm, table_hbm, out_hbm, idx_v, rows_v, gsem):
        wid = lax.axis_index("s") * _NC + lax.axis_index("c")
        base = wid * b_per_w
        pltpu.sync_copy(idx_hbm.at[pl.ds(base, b_per_w)], idx_v)
        copies = [
            pltpu.async_copy(
                table_hbm.at[idx_v.at[pl.ds(j * _CHUNK, _CHUNK)]],
                rows_v.at[pl.ds(j * _CHUNK, _CHUNK)],
                gsem,
            )
            for j in range(n_chunks)
        ]
        for c in copies:
            c.wait()
        pltpu.sync_copy(rows_v, out_hbm.at[pl.ds(base, b_per_w)])

    return k(idx, W)


# speedup vs baseline: 1.6567x; 1.6567x over previous
import jax
import jax.numpy as jnp
from jax import lax
from jax.experimental import pallas as pl
from jax.experimental.pallas import tpu as pltpu
from jax.experimental.pallas import tpu_sc as plsc

_NC = 2
_NS = 16
_NW = _NC * _NS
_NQ = 4


def kernel(idx, W):
    B = idx.shape[0]
    NZ = W.shape[1]
    b_per_w = B // _NW

    mesh = plsc.VectorSubcoreMesh(core_axis_name="c", subcore_axis_name="s")

    @pl.kernel(
        mesh=mesh,
        out_type=jax.ShapeDtypeStruct((B, NZ), W.dtype),
        compiler_params=pltpu.CompilerParams(needs_layout_passes=False),
        scratch_types=[
            pltpu.VMEM((b_per_w,), jnp.int32),
            pltpu.VMEM((b_per_w, NZ), jnp.float32),
            pltpu.SemaphoreType.DMA,
            pltpu.SemaphoreType.DMA,
            pltpu.SemaphoreType.DMA,
            pltpu.SemaphoreType.DMA,
        ],
    )
    def k(idx_hbm, table_hbm, out_hbm, idx_v, rows_v, s0, s1, s2, s3):
        wid = lax.axis_index("s") * _NC + lax.axis_index("c")
        base = wid * b_per_w
        pltpu.sync_copy(idx_hbm.at[pl.ds(base, b_per_w)], idx_v)
        sems = (s0, s1, s2, s3)

        @pl.loop(0, b_per_w, step=16)
        def _(i):
            vec = idx_v[pl.ds(i, 16)]
            for l in range(16):
                pltpu.async_copy(
                    table_hbm.at[pl.ds(vec[l], 1)],
                    rows_v.at[pl.ds(i + l, 1)], sems[l % _NQ])

        @pl.loop(0, b_per_w, step=_NQ)
        def _(r):
            for q in range(_NQ):
                pltpu.make_async_copy(
                    table_hbm.at[pl.ds(0, 1)],
                    rows_v.at[pl.ds(r + q, 1)], sems[q]).wait()

        pltpu.sync_copy(rows_v, out_hbm.at[pl.ds(base, b_per_w)])

    return k(idx, W)


# speedup vs baseline: 3.5989x; 2.1723x over previous
import jax
import jax.numpy as jnp
from jax import lax
from jax.experimental import pallas as pl
from jax.experimental.pallas import tpu as pltpu
from jax.experimental.pallas import tpu_sc as plsc

_NC = 2
_NS = 16
_NW = _NC * _NS
_LANES = 16
_WIN = 128


def kernel(idx, W):
    B = idx.shape[0]
    NZ = W.shape[1]
    b_per_w = B // _NW

    mesh = plsc.VectorSubcoreMesh(core_axis_name="c", subcore_axis_name="s")

    @pl.kernel(
        mesh=mesh,
        out_type=jax.ShapeDtypeStruct((NZ, B), W.dtype),
        compiler_params=pltpu.CompilerParams(needs_layout_passes=False),
        scratch_types=[
            pltpu.VMEM((b_per_w,), jnp.int32),
            pltpu.VMEM((NZ,), jnp.int32),
            pltpu.VMEM((_LANES, NZ, _WIN), jnp.float32),
            pltpu.VMEM((NZ, b_per_w), jnp.float32),
        ] + [pltpu.SemaphoreType.DMA] * _LANES,
    )
    def k(idx_hbm, table_hbm, out_hbm, idx_v, zid_v, win_v, out_v, *sems):
        wid = lax.axis_index("s") * _NC + lax.axis_index("c")
        base = wid * b_per_w
        pltpu.sync_copy(idx_hbm.at[pl.ds(base, b_per_w)], idx_v)

        lane = lax.iota(jnp.int32, _LANES)
        for h in range(NZ // _LANES):
            zid_v[pl.ds(h * _LANES, _LANES)] = lane + (h * _LANES)

        @pl.loop(0, b_per_w, step=_LANES)
        def _(i):
            vec = idx_v[pl.ds(i, _LANES)]
            copies = []
            for l in range(_LANES):
                win = (vec[l] // _WIN) * _WIN
                copies.append(pltpu.async_copy(
                    table_hbm.at[zid_v, pl.ds(win, _WIN)],
                    win_v.at[l], sems[l]))
            cols = vec % _WIN
            for l in range(_LANES):
                copies[l].wait()
                for h in range(NZ // _LANES):
                    zrows = lane + (h * _LANES)
                    vals = plsc.load_gather(
                        win_v.at[l], [zrows, jnp.full((_LANES,), cols[l])])
                    plsc.store_scatter(
                        out_v, [zrows, jnp.full((_LANES,), i + l)], vals)

        pltpu.sync_copy(out_v, out_hbm.at[:, pl.ds(base, b_per_w)])

    return k(idx, W.T).T
